# baseline (speedup 1.0000x reference)
import functools

import jax
import jax.numpy as jnp
from jax import lax
from jax.experimental import pallas as pl
from jax.experimental.pallas import tpu as pltpu
from jax.experimental.pallas import tpu_sc as plsc

N = 10000
E = 320000
D = 128
NC = 2
NS = 16
NW = NC * NS
EPW = E // NW
C = 128
NCH = 79
EP = NW * NCH * C
NPD = 10240
RPTD = NPD // NS
NPA = 10112
RPTA = NPA // NS

_mesh = plsc.VectorSubcoreMesh(core_axis_name="c", subcore_axis_name="s")


def _deg_body(col_hbm, zeros_hbm, ones_hbm, deg_out, colv, onesv, degsh):
    c = lax.axis_index("c")
    s = lax.axis_index("s")
    pltpu.sync_copy(zeros_hbm.at[pl.ds(0, RPTD)],
                    degsh.at[pl.ds(s * RPTD, RPTD)])
    pltpu.sync_copy(ones_hbm, onesv)
    pltpu.sync_copy(col_hbm.at[c, s], colv)
    plsc.subcore_barrier()

    def body(j, carry):
        pltpu.sync_copy(onesv, degsh.at[colv.at[j]], add=True)
        return carry

    lax.fori_loop(0, NCH, body, 0)
    plsc.subcore_barrier()
    pltpu.sync_copy(
        degsh.at[pl.ds(s * RPTD, RPTD)],
        deg_out.at[pl.ds(c * NPD + s * RPTD, RPTD)],
    )


def _agg_body(row_hbm, col_hbm, y_hbm, zeros_hbm, agg_out,
              colv, rowb0, rowb1, msgv, aggsh, sem_a, sem_ia, sem_ib):
    c = lax.axis_index("c")
    s = lax.axis_index("s")
    pltpu.sync_copy(zeros_hbm, aggsh.at[pl.ds(s * RPTA, RPTA)])
    pltpu.sync_copy(col_hbm.at[c, s], colv)
    plsc.subcore_barrier()

    pltpu.async_copy(row_hbm.at[c, s, pl.ds(0, C)], rowb0, sem_ia)
    pltpu.async_copy(row_hbm.at[c, s, pl.ds(C, C)], rowb1, sem_ib)

    def body(u, carry):
        j0 = 2 * u
        pltpu.make_async_copy(
            row_hbm.at[c, s, pl.ds(j0 * C, C)], rowb0, sem_ia).wait()
        pltpu.async_copy(y_hbm.at[rowb0], msgv, sem_a)
        pltpu.make_async_copy(y_hbm.at[rowb0], msgv, sem_a).wait()
        pltpu.async_copy(
            row_hbm.at[c, s, pl.ds((j0 + 2) * C, C)], rowb0, sem_ia)
        pltpu.sync_copy(msgv, aggsh.at[colv.at[j0]], add=True)
        pltpu.make_async_copy(
            row_hbm.at[c, s, pl.ds((j0 + 1) * C, C)], rowb1, sem_ib).wait()
        pltpu.async_copy(y_hbm.at[rowb1], msgv, sem_a)
        pltpu.make_async_copy(y_hbm.at[rowb1], msgv, sem_a).wait()
        pltpu.async_copy(
            row_hbm.at[c, s, pl.ds((j0 + 3) * C, C)], rowb1, sem_ib)
        pltpu.sync_copy(msgv, aggsh.at[colv.at[j0 + 1]], add=True)
        return carry

    lax.fori_loop(0, (NCH - 3) // 2, body, 0)
    j0 = NCH - 3
    pltpu.make_async_copy(
        row_hbm.at[c, s, pl.ds(j0 * C, C)], rowb0, sem_ia).wait()
    pltpu.async_copy(y_hbm.at[rowb0], msgv, sem_a)
    pltpu.make_async_copy(y_hbm.at[rowb0], msgv, sem_a).wait()
    pltpu.async_copy(
        row_hbm.at[c, s, pl.ds((j0 + 2) * C, C)], rowb0, sem_ia)
    pltpu.sync_copy(msgv, aggsh.at[colv.at[j0]], add=True)
    pltpu.make_async_copy(
        row_hbm.at[c, s, pl.ds((j0 + 1) * C, C)], rowb1, sem_ib).wait()
    pltpu.async_copy(y_hbm.at[rowb1], msgv, sem_a)
    pltpu.make_async_copy(y_hbm.at[rowb1], msgv, sem_a).wait()
    pltpu.sync_copy(msgv, aggsh.at[colv.at[j0 + 1]], add=True)
    pltpu.make_async_copy(
        row_hbm.at[c, s, pl.ds((j0 + 2) * C, C)], rowb0, sem_ia).wait()
    pltpu.async_copy(y_hbm.at[rowb0], msgv, sem_a)
    pltpu.make_async_copy(y_hbm.at[rowb0], msgv, sem_a).wait()
    pltpu.sync_copy(msgv, aggsh.at[colv.at[j0 + 2]], add=True)
    plsc.subcore_barrier()
    pltpu.sync_copy(
        aggsh.at[pl.ds(s * RPTA, RPTA)], agg_out.at[c, pl.ds(s * RPTA, RPTA)]
    )


def _make_deg_kernel(interpret=False):
    return functools.partial(
        pl.kernel,
        mesh=_mesh,
        out_type=jax.ShapeDtypeStruct((NC * NPD,), jnp.float32),
        scratch_types=[
            pltpu.VMEM((NCH, C), jnp.int32),
            pltpu.VMEM((C,), jnp.float32),
            pltpu.VMEM_SHARED((NPD,), jnp.float32),
        ],
        interpret=interpret,
    )(_deg_body)


def _make_agg_kernel(interpret=False):
    return functools.partial(
        pl.kernel,
        mesh=_mesh,
        out_type=jax.ShapeDtypeStruct((NC, NPA, D), jnp.float32),
        scratch_types=[
            pltpu.VMEM((NCH, C), jnp.int32),
            pltpu.VMEM((C,), jnp.int32),
            pltpu.VMEM((C,), jnp.int32),
            pltpu.VMEM((C, D), jnp.float32),
            pltpu.VMEM_SHARED((NPA, D), jnp.float32),
            pltpu.SemaphoreType.DMA,
            pltpu.SemaphoreType.DMA,
            pltpu.SemaphoreType.DMA,
        ],
        interpret=interpret,
    )(_agg_body)


_deg_kernel = _make_deg_kernel()
_agg_kernel = _make_agg_kernel()


def _scale_body(degt_ref, x_ref, y_ref):
    deg = degt_ref[:N, 0:1] + degt_ref[:N, 1:2] + 1.0
    dis = lax.rsqrt(deg)
    y_ref[...] = x_ref[...] * dis


def _dense_body(x_ref, agg_ref, degt_ref, cw_ref, cb_ref, gw_ref, gb_ref,
                gms_ref, l0w_ref, l0b_ref, l1w_ref, l1b_ref, out_ref):
    deg = degt_ref[:N, 0:1] + degt_ref[:N, 1:2] + 1.0
    dis = lax.rsqrt(deg)
    agg = agg_ref[0, :N, :] + agg_ref[1, :N, :]
    pre = dis * agg + (dis * dis) * x_ref[...]
    conv = jnp.dot(pre, cw_ref[...], preferred_element_type=jnp.float32)
    conv = conv + cb_ref[...]
    mean = jnp.mean(conv, axis=0, keepdims=True)
    t = conv - gms_ref[...] * mean
    var = jnp.mean(t * t, axis=0, keepdims=True)
    g = t * lax.rsqrt(var + 1e-5) * gw_ref[...] + gb_ref[...]
    g = jnp.maximum(g, 0.0)
    h1 = jnp.dot(g, l0w_ref[...], preferred_element_type=jnp.float32)
    h1 = jnp.maximum(h1 + l0b_ref[...], 0.0)
    out = jnp.dot(h1, l1w_ref[...], preferred_element_type=jnp.float32)
    out_ref[...] = out + l1b_ref[...]


def kernel(x, edge_index, conv_w, conv_b, gn_w, gn_b, gn_ms,
           lin0_w, lin0_b, lin1_w, lin1_b):
    pad = EP - E
    row = jnp.concatenate(
        [edge_index[0], jnp.zeros((pad,), jnp.int32)]).reshape(NC, NS, NCH * C)
    col = jnp.concatenate(
        [edge_index[1], jnp.full((pad,), N, jnp.int32)]).reshape(NC, NS, NCH, C)
    zeros_rows = jnp.zeros((RPTA, D), jnp.float32)
    zeros1d = jnp.zeros((NPD,), jnp.float32)
    ones1d = jnp.ones((C,), jnp.float32)

    deg = _deg_kernel(col, zeros1d, ones1d)
    degt = deg.reshape(NC, NPD).T

    y = pl.pallas_call(
        _scale_body,
        out_shape=jax.ShapeDtypeStruct((N, D), jnp.float32),
    )(degt, x)

    agg = _agg_kernel(row, col, y, zeros_rows)

    out = pl.pallas_call(
        _dense_body,
        out_shape=jax.ShapeDtypeStruct((N, D), jnp.float32),
    )(x, agg, degt, conv_w, conv_b.reshape(1, D), gn_w.reshape(1, D),
      gn_b.reshape(1, D), gn_ms.reshape(1, D), lin0_w,
      lin0_b.reshape(1, D), lin1_w, lin1_b.reshape(1, D))
    return out


# speedup vs baseline: 1.4621x; 1.4621x over previous
import functools

import jax
import jax.numpy as jnp
from jax import lax
from jax.experimental import pallas as pl
from jax.experimental.pallas import tpu as pltpu
from jax.experimental.pallas import tpu_sc as plsc

N = 10000
E = 320000
D = 128
NC = 2
NS = 16
NW = NC * NS
EPW = E // NW
C = 80
NCH = 125
EP = NW * NCH * C
NPD = 10240
RPTD = NPD // NS
NPA = 10240
RPTA = NPA // NS

_mesh = plsc.VectorSubcoreMesh(core_axis_name="c", subcore_axis_name="s")


def _deg_body(col_hbm, zeros_hbm, ones_hbm, deg_out, colv, onesv, degsh):
    c = lax.axis_index("c")
    s = lax.axis_index("s")
    pltpu.sync_copy(zeros_hbm.at[pl.ds(0, RPTD)],
                    degsh.at[pl.ds(s * RPTD, RPTD)])
    pltpu.sync_copy(ones_hbm, onesv)
    pltpu.sync_copy(col_hbm.at[c, s], colv)
    plsc.subcore_barrier()

    def body(j, carry):
        pltpu.sync_copy(onesv, degsh.at[colv.at[j]], add=True)
        return carry

    lax.fori_loop(0, NCH, body, 0)
    plsc.subcore_barrier()
    pltpu.sync_copy(
        degsh.at[pl.ds(s * RPTD, RPTD)],
        deg_out.at[pl.ds(c * NPD + s * RPTD, RPTD)],
    )


def _agg_body(row_hbm, col_hbm, y_hbm, zeros_hbm, agg_out,
              rowv, colv, msgv, aggsh, sem):
    c = lax.axis_index("c")
    s = lax.axis_index("s")
    pltpu.sync_copy(zeros_hbm, aggsh.at[pl.ds(s * RPTA, RPTA)])
    pltpu.sync_copy(row_hbm.at[c, s], rowv)
    pltpu.sync_copy(col_hbm.at[c, s], colv)
    plsc.subcore_barrier()

    def body(j, carry):
        pltpu.async_copy(y_hbm.at[rowv.at[j]], msgv, sem).wait()
        pltpu.sync_copy(msgv, aggsh.at[colv.at[j]], add=True)
        return carry

    lax.fori_loop(0, NCH, body, 0)
    plsc.subcore_barrier()
    pltpu.sync_copy(
        aggsh.at[pl.ds(s * RPTA, RPTA)], agg_out.at[c, pl.ds(s * RPTA, RPTA)]
    )


def _make_deg_kernel(interpret=False):
    return functools.partial(
        pl.kernel,
        mesh=_mesh,
        out_type=jax.ShapeDtypeStruct((NC * NPD,), jnp.float32),
        scratch_types=[
            pltpu.VMEM((NCH, C), jnp.int32),
            pltpu.VMEM((C,), jnp.float32),
            pltpu.VMEM_SHARED((NPD,), jnp.float32),
        ],
        interpret=interpret,
    )(_deg_body)


def _make_agg_kernel(interpret=False):
    return functools.partial(
        pl.kernel,
        mesh=_mesh,
        out_type=jax.ShapeDtypeStruct((NC, NPA, D), jnp.float32),
        scratch_types=[
            pltpu.VMEM((NCH, C), jnp.int32),
            pltpu.VMEM((NCH, C), jnp.int32),
            pltpu.VMEM((C, D), jnp.float32),
            pltpu.VMEM_SHARED((NPA, D), jnp.float32),
            pltpu.SemaphoreType.DMA,
        ],
        interpret=interpret,
    )(_agg_body)


_deg_kernel = _make_deg_kernel()
_agg_kernel = _make_agg_kernel()


def _scale_body(degt_ref, x_ref, y_ref):
    deg = degt_ref[:N, 0:1] + degt_ref[:N, 1:2] + 1.0
    dis = lax.rsqrt(deg)
    y_ref[...] = x_ref[...] * dis


def _dense_body(x_ref, agg_ref, degt_ref, cw_ref, cb_ref, gw_ref, gb_ref,
                gms_ref, l0w_ref, l0b_ref, l1w_ref, l1b_ref, out_ref):
    deg = degt_ref[:N, 0:1] + degt_ref[:N, 1:2] + 1.0
    dis = lax.rsqrt(deg)
    agg = agg_ref[0, :N, :] + agg_ref[1, :N, :]
    pre = dis * agg + (dis * dis) * x_ref[...]
    conv = jnp.dot(pre, cw_ref[...], preferred_element_type=jnp.float32)
    conv = conv + cb_ref[...]
    mean = jnp.mean(conv, axis=0, keepdims=True)
    t = conv - gms_ref[...] * mean
    var = jnp.mean(t * t, axis=0, keepdims=True)
    g = t * lax.rsqrt(var + 1e-5) * gw_ref[...] + gb_ref[...]
    g = jnp.maximum(g, 0.0)
    h1 = jnp.dot(g, l0w_ref[...], preferred_element_type=jnp.float32)
    h1 = jnp.maximum(h1 + l0b_ref[...], 0.0)
    out = jnp.dot(h1, l1w_ref[...], preferred_element_type=jnp.float32)
    out_ref[...] = out + l1b_ref[...]


def kernel(x, edge_index, conv_w, conv_b, gn_w, gn_b, gn_ms,
           lin0_w, lin0_b, lin1_w, lin1_b):
    pad = EP - E
    row = jnp.concatenate(
        [edge_index[0], jnp.zeros((pad,), jnp.int32)]).reshape(NC, NS, NCH, C)
    col = jnp.concatenate(
        [edge_index[1], jnp.full((pad,), N, jnp.int32)]).reshape(NC, NS, NCH, C)
    zeros_rows = jnp.zeros((RPTA, D), jnp.float32)
    zeros1d = jnp.zeros((NPD,), jnp.float32)
    ones1d = jnp.ones((C,), jnp.float32)

    deg = _deg_kernel(col, zeros1d, ones1d)
    degt = deg.reshape(NC, NPD).T

    y = pl.pallas_call(
        _scale_body,
        out_shape=jax.ShapeDtypeStruct((N, D), jnp.float32),
    )(degt, x)

    agg = _agg_kernel(row, col, y, zeros_rows)

    out = pl.pallas_call(
        _dense_body,
        out_shape=jax.ShapeDtypeStruct((N, D), jnp.float32),
    )(x, agg, degt, conv_w, conv_b.reshape(1, D), gn_w.reshape(1, D),
      gn_b.reshape(1, D), gn_ms.reshape(1, D), lin0_w,
      lin0_b.reshape(1, D), lin1_w, lin1_b.reshape(1, D))
    return out
